# Initial kernel scaffold; baseline (speedup 1.0000x reference)
import functools

import jax
import jax.numpy as jnp
from jax import lax
from jax.experimental import pallas as pl
from jax.experimental.pallas import tpu as pltpu
from jax.experimental.pallas import tpu_sc as plsc

VOCAB = 100000
EMBED_DIM = 128
BATCH = 4096
HIST = 200

N = BATCH * HIST
NC, NS = 2, 16
NW = NC * NS
B_PER_W = N // NW
CHUNK = 400
NCHUNK = B_PER_W // CHUNK


def _make_kernel():
  mesh = plsc.VectorSubcoreMesh(core_axis_name="c", subcore_axis_name="s")

  @functools.partial(
      pl.kernel,
      mesh=mesh,
      out_type=jax.ShapeDtypeStruct((N, EMBED_DIM), jnp.float32),
      scratch_types=[
          pltpu.VMEM((B_PER_W,), jnp.int32),
          pltpu.VMEM((2, CHUNK, EMBED_DIM), jnp.float32),
          pltpu.SemaphoreType.DMA,
          pltpu.SemaphoreType.DMA,
      ],
  )
  def k(x_hbm, table_hbm, out_hbm, idx_v, rows_v, gsem, wsem):
    wid = lax.axis_index("s") * NC + lax.axis_index("c")
    base = wid * B_PER_W
    pltpu.sync_copy(x_hbm.at[pl.ds(base, B_PER_W)], idx_v)

    pltpu.async_copy(
        table_hbm.at[idx_v.at[pl.ds(0, CHUNK)]], rows_v.at[0], gsem)

    def body(g, _):
      buf = lax.rem(g, 2)
      nxt = 1 - buf
      @pl.when(g + 1 < NCHUNK)
      def _():
        pltpu.async_copy(
            table_hbm.at[idx_v.at[pl.ds((g + 1) * CHUNK, CHUNK)]],
            rows_v.at[nxt], gsem)
      pltpu.make_async_copy(
          table_hbm.at[idx_v.at[pl.ds(g * CHUNK, CHUNK)]],
          rows_v.at[buf], gsem).wait()
      pltpu.async_copy(
          rows_v.at[buf], out_hbm.at[pl.ds(base + g * CHUNK, CHUNK)], wsem)
      @pl.when(g >= 1)
      def _():
        pltpu.make_async_copy(
            rows_v.at[nxt],
            out_hbm.at[pl.ds(base + (g - 1) * CHUNK, CHUNK)], wsem).wait()
      return ()

    pl.loop(0, NCHUNK)(body)
    pltpu.make_async_copy(
        rows_v.at[(NCHUNK - 1) % 2],
        out_hbm.at[pl.ds(base + (NCHUNK - 1) * CHUNK, CHUNK)], wsem).wait()

  return k


_gather_kernel = _make_kernel()


@jax.jit
def kernel(x, table):
  flat = x.reshape(N).astype(jnp.int32)
  out = _gather_kernel(flat, table)
  return out.reshape(BATCH, HIST, EMBED_DIM)


# speedup vs baseline: 9.1931x; 9.1931x over previous
import functools

import jax
import jax.numpy as jnp
from jax import lax
from jax.experimental import pallas as pl
from jax.experimental.pallas import tpu as pltpu
from jax.experimental.pallas import tpu_sc as plsc

VOCAB = 100000
EMBED_DIM = 128
BATCH = 4096
HIST = 200

N = BATCH * HIST
NC, NS = 2, 16
NW = NC * NS
B_PER_W = N // NW
IW = 128
NIV = B_PER_W // IW
SUB = 2
R = SUB * IW
NG = B_PER_W // R


def _make_kernel():
  mesh = plsc.VectorSubcoreMesh(core_axis_name="c", subcore_axis_name="s")

  @functools.partial(
      pl.kernel,
      mesh=mesh,
      out_type=jax.ShapeDtypeStruct((N, EMBED_DIM), jnp.float32),
      scratch_types=[
          pltpu.VMEM((NIV, IW), jnp.int32),
          pltpu.VMEM((2, R, EMBED_DIM), jnp.float32),
          pltpu.SemaphoreType.DMA,
          pltpu.SemaphoreType.DMA,
          pltpu.SemaphoreType.DMA,
          pltpu.SemaphoreType.DMA,
      ],
  )
  def k(x_hbm, table_hbm, out_hbm, idx_v, rows_v, gsem0, gsem1, wsem0, wsem1):
    wid = lax.axis_index("s") * NC + lax.axis_index("c")
    base = wid * B_PER_W
    gsem = (gsem0, gsem1)
    wsem = (wsem0, wsem1)

    pltpu.sync_copy(x_hbm.at[pl.ds(wid * NIV, NIV)], idx_v.at[...])

    def gather(g, b):
      for j in range(SUB):
        pltpu.async_copy(
            table_hbm.at[idx_v.at[SUB * g + j]],
            rows_v.at[b, pl.ds(j * IW, IW)], gsem[b])

    def gather_wait(g, b):
      for j in range(SUB):
        pltpu.make_async_copy(
            table_hbm.at[idx_v.at[SUB * g + j]],
            rows_v.at[b, pl.ds(j * IW, IW)], gsem[b]).wait()

    def write(g, b):
      pltpu.async_copy(
          rows_v.at[b], out_hbm.at[pl.ds(base + g * R, R)], wsem[b])

    def write_wait(g, b):
      pltpu.make_async_copy(
          rows_v.at[b], out_hbm.at[pl.ds(base + g * R, R)], wsem[b]).wait()

    gather(0, 0)

    def body(g):
      for b in range(2):
        gg = g + b
        o = 1 - b
        @pl.when(jnp.logical_and(gg >= 1, gg + 1 < NG))
        def _():
          write_wait(gg - 1, o)
        @pl.when(gg + 1 < NG)
        def _():
          gather(gg + 1, o)
        gather_wait(gg, b)
        write(gg, b)

    pl.loop(0, NG, step=2)(body)

    write_wait(NG - 2, 0)
    write_wait(NG - 1, 1)

  return k


_gather_kernel = _make_kernel()


@jax.jit
def kernel(x, table):
  flat = x.reshape(N // IW, IW).astype(jnp.int32)
  out = _gather_kernel(flat, table)
  return out.reshape(BATCH, HIST, EMBED_DIM)


# speedup vs baseline: 9.2058x; 1.0014x over previous
import functools

import jax
import jax.numpy as jnp
from jax import lax
from jax.experimental import pallas as pl
from jax.experimental.pallas import tpu as pltpu
from jax.experimental.pallas import tpu_sc as plsc

VOCAB = 100000
EMBED_DIM = 128
BATCH = 4096
HIST = 200

N = BATCH * HIST
NC, NS = 2, 16
NW = NC * NS
B_PER_W = N // NW
IW = 128
NIV = B_PER_W // IW
R = IW
NG = B_PER_W // R
NBUF = 4


def _make_kernel():
  mesh = plsc.VectorSubcoreMesh(core_axis_name="c", subcore_axis_name="s")

  @functools.partial(
      pl.kernel,
      mesh=mesh,
      out_type=jax.ShapeDtypeStruct((N, EMBED_DIM), jnp.float32),
      scratch_types=[
          pltpu.VMEM((NIV, IW), jnp.int32),
          pltpu.VMEM((NBUF, R, EMBED_DIM), jnp.float32),
          [pltpu.SemaphoreType.DMA] * NBUF,
          [pltpu.SemaphoreType.DMA] * NBUF,
      ],
  )
  def k(x_hbm, table_hbm, out_hbm, idx_v, rows_v, gsem, wsem):
    wid = lax.axis_index("s") * NC + lax.axis_index("c")
    base = wid * B_PER_W

    pltpu.sync_copy(x_hbm.at[pl.ds(wid * NIV, NIV)], idx_v.at[...])

    def gather(g, b):
      pltpu.async_copy(table_hbm.at[idx_v.at[g]], rows_v.at[b], gsem[b])

    def gather_wait(g, b):
      pltpu.make_async_copy(
          table_hbm.at[idx_v.at[g]], rows_v.at[b], gsem[b]).wait()

    def write(g, b):
      pltpu.async_copy(
          rows_v.at[b], out_hbm.at[pl.ds(base + g * R, R)], wsem[b])

    def write_wait(g, b):
      pltpu.make_async_copy(
          rows_v.at[b], out_hbm.at[pl.ds(base + g * R, R)], wsem[b]).wait()

    for g in range(NBUF - 1):
      gather(g, g)

    def body(g):
      for b in range(NBUF):
        gg = g + b
        pb = (b + NBUF - 1) % NBUF
        @pl.when(gg >= 1)
        def _():
          write_wait(gg - 1, pb)
        @pl.when(gg + NBUF - 1 < NG)
        def _():
          gather(gg + NBUF - 1, pb)
        gather_wait(gg, b)
        write(gg, b)

    pl.loop(0, NG, step=NBUF)(body)

    write_wait(NG - 1, (NG - 1) % NBUF)

  return k


_gather_kernel = _make_kernel()


@jax.jit
def kernel(x, table):
  flat = x.reshape(N // IW, IW).astype(jnp.int32)
  out = _gather_kernel(flat, table)
  return out.reshape(BATCH, HIST, EMBED_DIM)


# speedup vs baseline: 9.2170x; 1.0012x over previous
import functools

import jax
import jax.numpy as jnp
from jax import lax
from jax.experimental import pallas as pl
from jax.experimental.pallas import tpu as pltpu
from jax.experimental.pallas import tpu_sc as plsc

VOCAB = 100000
EMBED_DIM = 128
BATCH = 4096
HIST = 200

N = BATCH * HIST
NC, NS = 2, 16
NW = NC * NS
B_PER_W = N // NW
IW = 128
NIV = B_PER_W // IW
R = IW
NG = B_PER_W // R
NBUF = 5


def _make_kernel():
  mesh = plsc.VectorSubcoreMesh(core_axis_name="c", subcore_axis_name="s")

  @functools.partial(
      pl.kernel,
      mesh=mesh,
      out_type=jax.ShapeDtypeStruct((N, EMBED_DIM), jnp.float32),
      scratch_types=[
          pltpu.VMEM((NIV, IW), jnp.int32),
          pltpu.VMEM((NBUF, R, EMBED_DIM), jnp.float32),
          [pltpu.SemaphoreType.DMA] * NBUF,
          [pltpu.SemaphoreType.DMA] * NBUF,
      ],
  )
  def k(x_hbm, table_hbm, out_hbm, idx_v, rows_v, gsem, wsem):
    wid = lax.axis_index("s") * NC + lax.axis_index("c")
    base = wid * B_PER_W

    pltpu.sync_copy(x_hbm.at[pl.ds(wid * NIV, NIV)], idx_v.at[...])

    def gather(g, b):
      pltpu.async_copy(table_hbm.at[idx_v.at[g]], rows_v.at[b], gsem[b])

    def gather_wait(g, b):
      pltpu.make_async_copy(
          table_hbm.at[idx_v.at[g]], rows_v.at[b], gsem[b]).wait()

    def write(g, b):
      pltpu.async_copy(
          rows_v.at[b], out_hbm.at[pl.ds(base + g * R, R)], wsem[b])

    def write_wait(g, b):
      pltpu.make_async_copy(
          rows_v.at[b], out_hbm.at[pl.ds(base + g * R, R)], wsem[b]).wait()

    for g in range(NBUF - 1):
      gather(g, g)

    def body(g):
      for b in range(NBUF):
        gg = g + b
        pb = (b + NBUF - 1) % NBUF
        @pl.when(gg >= 1)
        def _():
          write_wait(gg - 1, pb)
        @pl.when(gg + NBUF - 1 < NG)
        def _():
          gather(gg + NBUF - 1, pb)
        gather_wait(gg, b)
        write(gg, b)

    pl.loop(0, NG, step=NBUF)(body)

    write_wait(NG - 1, (NG - 1) % NBUF)

  return k


_gather_kernel = _make_kernel()


@jax.jit
def kernel(x, table):
  flat = x.reshape(N // IW, IW).astype(jnp.int32)
  out = _gather_kernel(flat, table)
  return out.reshape(BATCH, HIST, EMBED_DIM)
